# Initial kernel scaffold; baseline (speedup 1.0000x reference)
import jax
import jax.numpy as jnp
from jax.experimental import pallas as pl


def kernel(x, PA, Wdown, bdown, gdown, betdown, Wsub, bsub, gsub, betsub, Wedge, gedge, betedge, gbn, bbn):
    raise NotImplementedError("write your pallas kernel here")


# speedup vs baseline: 1.0132x; 1.0132x over previous
import jax
import jax.numpy as jnp
from jax.experimental import pallas as pl

_N, _C, _T, _V = 128, 64, 64, 25
_L, _S, _INTER = 3, 3, 16
_INV = (1.0 + 1e-5) ** -0.5
_NEG = -1e30


def _tc_body(x2_ref, x3_ref, wd_ref, bd_ref, pacat_ref, wblk_ref,
             mavg_ref, sb_ref, wa_ref, wb_ref, bb_ref, gout_ref, bout_ref,
             out_ref):
    xn = x2_ref[0]
    xd = jnp.dot(wd_ref[...], xn, preferred_element_type=jnp.float32)
    xd = jnp.maximum(xd + bd_ref[...], 0.0)

    zsum = [None, None, None]
    esum = None
    for i in range(_L):
        xdi = xd[i * _INTER:(i + 1) * _INTER, :]
        y = xdi.T
        y3 = y.reshape(_T, _V, _INTER)
        y3t = jnp.transpose(y3, (0, 2, 1))
        xdb = y3t.reshape(_T * _INTER, _V)

        xbar = jnp.dot(mavg_ref[...], xdb,
                       preferred_element_type=jnp.float32)
        zpre = jnp.dot(xdb, pacat_ref[i],
                       preferred_element_type=jnp.float32)
        for j in range(_S):
            zj = zpre[:, j * _V:(j + 1) * _V]
            parts = []
            for tb in range(8):
                chunk = zj[tb * 128:(tb + 1) * 128, :]
                parts.append(jnp.dot(wblk_ref[i, j], chunk,
                                     preferred_element_type=jnp.float32))
            zja = jnp.concatenate(parts, axis=0)
            z3 = zja.reshape(_T, _INTER, _V)
            z3t = jnp.transpose(z3, (1, 0, 2))
            zjo = z3t.reshape(_INTER * _T, _V)
            zsum[j] = zjo if zsum[j] is None else zsum[j] + zjo

        xx = jnp.sum(xbar * xbar, axis=0)
        gram = jax.lax.dot_general(xbar, xbar, (((0,), (0,)), ((), ())),
                                   preferred_element_type=jnp.float32)
        scoret = 2.0 * gram - xx[:, None]
        s = scoret
        for _ in range(4):
            m = jnp.max(s, axis=0, keepdims=True)
            s = jnp.where(s == m, _NEG, s)
        thresh = jnp.max(s, axis=0, keepdims=True)
        maskt = scoret >= thresh
        a2 = jnp.dot(wa_ref[i], xbar,
                     preferred_element_type=jnp.float32)
        b2 = jnp.dot(wb_ref[i], xbar,
                     preferred_element_type=jnp.float32)
        b2 = b2 + bb_ref[i]
        amax = None
        for u in range(_V):
            cand = jnp.where(maskt[u:u + 1, :], a2[:, u:u + 1], _NEG)
            amax = cand if amax is None else jnp.maximum(amax, cand)
        e2 = amax + b2
        e2 = jnp.where(e2 > 0, e2, 0.2 * e2)
        esum = e2 if esum is None else esum + e2

    eb = jnp.dot(sb_ref[...], esum,
                 preferred_element_type=jnp.float32)
    zcat = jnp.concatenate(zsum + [eb], axis=0)
    out = zcat * gout_ref[...] + bout_ref[...] + x3_ref[0]
    out_ref[0] = jnp.maximum(out, 0.0)


def _run(x2, x3, Wd, bd, PAcat, Wblk, Mavg, Sb, WA, WB, bb, gout, bout,
         interpret=False):
    return pl.pallas_call(
        _tc_body,
        grid=(_N,),
        in_specs=[
            pl.BlockSpec((1, _C, _T * _V), lambda n: (n, 0, 0)),
            pl.BlockSpec((1, _C * _T, _V), lambda n: (n, 0, 0)),
            pl.BlockSpec((_L * _INTER, _C), lambda n: (0, 0)),
            pl.BlockSpec((_L * _INTER, 1), lambda n: (0, 0)),
            pl.BlockSpec((_L, _V, _S * _V), lambda n: (0, 0, 0)),
            pl.BlockSpec((_L, _S, 128, 128), lambda n: (0, 0, 0, 0)),
            pl.BlockSpec((_INTER, _T * _INTER), lambda n: (0, 0)),
            pl.BlockSpec((_INTER * _T, _INTER), lambda n: (0, 0)),
            pl.BlockSpec((_L, _INTER, _INTER), lambda n: (0, 0, 0)),
            pl.BlockSpec((_L, _INTER, _INTER), lambda n: (0, 0, 0)),
            pl.BlockSpec((_L, _INTER, 1), lambda n: (0, 0, 0)),
            pl.BlockSpec((_C * _T, 1), lambda n: (0, 0)),
            pl.BlockSpec((_C * _T, 1), lambda n: (0, 0)),
        ],
        out_specs=pl.BlockSpec((1, _C * _T, _V), lambda n: (n, 0, 0)),
        out_shape=jax.ShapeDtypeStruct((_N, _C * _T, _V), jnp.float32),
        interpret=interpret,
    )(x2, x3, Wd, bd, PAcat, Wblk, Mavg, Sb, WA, WB, bb, gout, bout)


def _prep(x, PA, Wdown, bdown, gdown, betdown, Wsub, bsub, gsub, betsub,
          Wedge, gedge, betedge, gbn, bbn):
    x2 = x.reshape(_N, _C, _T * _V)
    x3 = x.reshape(_N, _C * _T, _V)
    sdown = gdown * _INV
    Wd = (Wdown * sdown[:, :, None]).reshape(_L * _INTER, _C)
    bd = (bdown * sdown + betdown).reshape(_L * _INTER, 1)
    PAcat = jnp.transpose(PA, (0, 3, 1, 2)).reshape(_L, _V, _S * _V)
    ssub = gsub * _INV
    Wsubf = Wsub * ssub[..., None]
    eye8 = jnp.eye(8, dtype=x.dtype)
    Wblk = jnp.einsum('tu,ijoc->ijtouc', eye8, Wsubf).reshape(
        _L, _S, 128, 128)
    Mavg = jnp.tile(jnp.eye(_INTER, dtype=x.dtype) / _T, (1, _T))
    Sb = jnp.repeat(jnp.eye(_INTER, dtype=x.dtype), _T, axis=0)
    sedge = gedge * _INV
    W1 = Wedge[:, :, :_INTER]
    W2 = Wedge[:, :, _INTER:]
    WA = W1 * sedge[..., None]
    WB = (W2 - W1) * sedge[..., None]
    bb = betedge.reshape(_L, _INTER, 1)
    gout = jnp.repeat(gbn * _INV, _T)[:, None]
    bsubf = (bsub * ssub + betsub)
    bias_z = jnp.sum(bsubf, axis=0)
    bias_full = jnp.concatenate(
        [bias_z.reshape(_S * _INTER), jnp.zeros((_INTER,), x.dtype)])
    bout = jnp.repeat(bias_full * (gbn * _INV) + bbn, _T)[:, None]
    return x2, x3, Wd, bd, PAcat, Wblk, Mavg, Sb, WA, WB, bb, gout, bout


@jax.jit
def kernel(x, PA, Wdown, bdown, gdown, betdown, Wsub, bsub, gsub, betsub,
           Wedge, gedge, betedge, gbn, bbn):
    args = _prep(x, PA, Wdown, bdown, gdown, betdown, Wsub, bsub, gsub,
                 betsub, Wedge, gedge, betedge, gbn, bbn)
    out3 = _run(*args)
    return out3.reshape(_N, _C, _T, _V)


# speedup vs baseline: 1.3828x; 1.3649x over previous
import jax
import jax.numpy as jnp
from jax.experimental import pallas as pl

_N, _C, _T, _V = 128, 64, 64, 25
_L, _S, _INTER = 3, 3, 16
_INV = (1.0 + 1e-5) ** -0.5
_NEG = -1e30


def _tc_body(x2_ref, x3_ref, wd_ref, bd_ref, pacat_ref, wblk_ref,
             mavg_ref, sb_ref, wa_ref, wb_ref, bb_ref, gout_ref, bout_ref,
             out_ref):
    xn = x2_ref[0]
    xd = jnp.dot(wd_ref[...], xn, preferred_element_type=jnp.float32)
    xd = jnp.maximum(xd + bd_ref[...], 0.0)

    yall = xd.T
    y3 = yall.reshape(_T, _V, _L * _INTER)
    y3t = jnp.transpose(y3, (0, 2, 1))

    zsum = [None, None, None]
    esum = None
    for i in range(_L):
        xdb = y3t[:, i * _INTER:(i + 1) * _INTER, :].reshape(
            _T * _INTER, _V)

        xbar = jnp.dot(mavg_ref[...], xdb,
                       preferred_element_type=jnp.float32)
        zpre = jnp.dot(xdb, pacat_ref[i],
                       preferred_element_type=jnp.float32)
        for j in range(_S):
            zj = zpre[:, j * _V:(j + 1) * _V]
            parts = []
            for tb in range(8):
                chunk = zj[tb * 128:(tb + 1) * 128, :]
                parts.append(jnp.dot(wblk_ref[i, j], chunk,
                                     preferred_element_type=jnp.float32))
            zja = jnp.concatenate(parts, axis=0)
            zsum[j] = zja if zsum[j] is None else zsum[j] + zja

        xx = jnp.sum(xbar * xbar, axis=0)
        gram = jax.lax.dot_general(xbar, xbar, (((0,), (0,)), ((), ())),
                                   preferred_element_type=jnp.float32)
        scoret = 2.0 * gram - xx[:, None]
        s = scoret
        for _ in range(4):
            m = jnp.max(s, axis=0, keepdims=True)
            s = jnp.where(s == m, _NEG, s)
        thresh = jnp.max(s, axis=0, keepdims=True)
        maskt = scoret >= thresh
        a2 = jnp.dot(wa_ref[i], xbar,
                     preferred_element_type=jnp.float32)
        b2 = jnp.dot(wb_ref[i], xbar,
                     preferred_element_type=jnp.float32)
        b2 = b2 + bb_ref[i]
        amax = None
        for u in range(_V):
            cand = jnp.where(maskt[u:u + 1, :], a2[:, u:u + 1], _NEG)
            amax = cand if amax is None else jnp.maximum(amax, cand)
        e2 = amax + b2
        e2 = jnp.where(e2 > 0, e2, 0.2 * e2)
        esum = e2 if esum is None else esum + e2

    eb = jnp.dot(sb_ref[...], esum,
                 preferred_element_type=jnp.float32)
    zot = []
    for j in range(_S):
        z3 = zsum[j].reshape(_T, _INTER, _V)
        z3t = jnp.transpose(z3, (1, 0, 2))
        zot.append(z3t.reshape(_INTER * _T, _V))
    zcat = jnp.concatenate(zot + [eb], axis=0)
    out = zcat * gout_ref[...] + bout_ref[...] + x3_ref[0]
    out_ref[0] = jnp.maximum(out, 0.0)


def _run(x2, x3, Wd, bd, PAcat, Wblk, Mavg, Sb, WA, WB, bb, gout, bout,
         interpret=False):
    return pl.pallas_call(
        _tc_body,
        grid=(_N,),
        in_specs=[
            pl.BlockSpec((1, _C, _T * _V), lambda n: (n, 0, 0)),
            pl.BlockSpec((1, _C * _T, _V), lambda n: (n, 0, 0)),
            pl.BlockSpec((_L * _INTER, _C), lambda n: (0, 0)),
            pl.BlockSpec((_L * _INTER, 1), lambda n: (0, 0)),
            pl.BlockSpec((_L, _V, _S * _V), lambda n: (0, 0, 0)),
            pl.BlockSpec((_L, _S, 128, 128), lambda n: (0, 0, 0, 0)),
            pl.BlockSpec((_INTER, _T * _INTER), lambda n: (0, 0)),
            pl.BlockSpec((_INTER * _T, _INTER), lambda n: (0, 0)),
            pl.BlockSpec((_L, _INTER, _INTER), lambda n: (0, 0, 0)),
            pl.BlockSpec((_L, _INTER, _INTER), lambda n: (0, 0, 0)),
            pl.BlockSpec((_L, _INTER, 1), lambda n: (0, 0, 0)),
            pl.BlockSpec((_C * _T, 1), lambda n: (0, 0)),
            pl.BlockSpec((_C * _T, 1), lambda n: (0, 0)),
        ],
        out_specs=pl.BlockSpec((1, _C * _T, _V), lambda n: (n, 0, 0)),
        out_shape=jax.ShapeDtypeStruct((_N, _C * _T, _V), jnp.float32),
        interpret=interpret,
    )(x2, x3, Wd, bd, PAcat, Wblk, Mavg, Sb, WA, WB, bb, gout, bout)


def _prep(x, PA, Wdown, bdown, gdown, betdown, Wsub, bsub, gsub, betsub,
          Wedge, gedge, betedge, gbn, bbn):
    x2 = x.reshape(_N, _C, _T * _V)
    x3 = x.reshape(_N, _C * _T, _V)
    sdown = gdown * _INV
    Wd = (Wdown * sdown[:, :, None]).reshape(_L * _INTER, _C)
    bd = (bdown * sdown + betdown).reshape(_L * _INTER, 1)
    PAcat = jnp.transpose(PA, (0, 3, 1, 2)).reshape(_L, _V, _S * _V)
    ssub = gsub * _INV
    Wsubf = Wsub * ssub[..., None]
    eye8 = jnp.eye(8, dtype=x.dtype)
    Wblk = jnp.einsum('tu,ijoc->ijtouc', eye8, Wsubf).reshape(
        _L, _S, 128, 128)
    Mavg = jnp.tile(jnp.eye(_INTER, dtype=x.dtype) / _T, (1, _T))
    Sb = jnp.repeat(jnp.eye(_INTER, dtype=x.dtype), _T, axis=0)
    sedge = gedge * _INV
    W1 = Wedge[:, :, :_INTER]
    W2 = Wedge[:, :, _INTER:]
    WA = W1 * sedge[..., None]
    WB = (W2 - W1) * sedge[..., None]
    bb = betedge.reshape(_L, _INTER, 1)
    gout = jnp.repeat(gbn * _INV, _T)[:, None]
    bsubf = (bsub * ssub + betsub)
    bias_z = jnp.sum(bsubf, axis=0)
    bias_full = jnp.concatenate(
        [bias_z.reshape(_S * _INTER), jnp.zeros((_INTER,), x.dtype)])
    bout = jnp.repeat(bias_full * (gbn * _INV) + bbn, _T)[:, None]
    return x2, x3, Wd, bd, PAcat, Wblk, Mavg, Sb, WA, WB, bb, gout, bout


@jax.jit
def kernel(x, PA, Wdown, bdown, gdown, betdown, Wsub, bsub, gsub, betsub,
           Wedge, gedge, betedge, gbn, bbn):
    args = _prep(x, PA, Wdown, bdown, gdown, betdown, Wsub, bsub, gsub,
                 betsub, Wedge, gedge, betedge, gbn, bbn)
    out3 = _run(*args)
    return out3.reshape(_N, _C, _T, _V)
